# Initial kernel scaffold; baseline (speedup 1.0000x reference)
import jax
import jax.numpy as jnp
from jax.experimental import pallas as pl


def kernel(grid_ids, special_pos, table, unk_emb):
    raise NotImplementedError("write your pallas kernel here")


# speedup vs baseline: 1.9779x; 1.9779x over previous
import functools

import jax
import jax.numpy as jnp
from jax import lax
from jax.experimental import pallas as pl
from jax.experimental.pallas import tpu as pltpu
from jax.experimental.pallas import tpu_sc as plsc

_UNKNOWN = 1
_LANES = 16
_IDXW = 128


@functools.lru_cache(maxsize=None)
def _build(n, d, chunk):
    info = plsc.get_sparse_core_info()
    nw = info.num_cores * info.num_subcores
    assert n % (nw * chunk) == 0
    per_w = n // nw
    nchunk = per_w // chunk
    ndma = chunk // _IDXW
    ngrp = chunk // _LANES
    gpr = _IDXW // _LANES

    mesh = plsc.VectorSubcoreMesh(core_axis_name="c", subcore_axis_name="s")

    @functools.partial(
        pl.kernel,
        mesh=mesh,
        compiler_params=pltpu.CompilerParams(
            needs_layout_passes=False, use_tc_tiling_on_sc=False),
        out_type=jax.ShapeDtypeStruct((n, d), jnp.float32),
        scratch_types=[
            pltpu.VMEM((chunk,), jnp.int32),
            pltpu.VMEM((chunk,), jnp.int32),
            pltpu.VMEM((ndma, _IDXW), jnp.int32),
            pltpu.VMEM((chunk, d), jnp.float32),
            pltpu.VMEM((d,), jnp.float32),
            pltpu.SemaphoreType.DMA,
        ],
    )
    def emb(gids_hbm, spec_hbm, table_hbm, unk_hbm, out_hbm,
            gids_v, flag_v, ids_v, rows_v, unk_v, sem):
        wid = lax.axis_index("s") * info.num_cores + lax.axis_index("c")
        base = wid * per_w
        pltpu.sync_copy(unk_hbm, unk_v)

        def chunk_body(ci, carry):
            off = base + ci * chunk
            pltpu.sync_copy(gids_hbm.at[pl.ds(off, chunk)], gids_v)
            pltpu.sync_copy(spec_hbm.at[pl.ds(off, chunk)], flag_v)

            def grp(j, acc):
                g = gids_v[pl.ds(j * _LANES, _LANES)]
                s = flag_v[pl.ds(j * _LANES, _LANES)]
                unk = g == _UNKNOWN
                unk_i = jnp.where(unk, 1, 0)
                masked = jnp.where(jnp.logical_or(unk, s != 0), 0, g)
                ids_v[j // gpr, pl.ds((j % gpr) * _LANES, _LANES)] = masked
                flag_v[pl.ds(j * _LANES, _LANES)] = unk_i
                return acc | unk_i

            acc = lax.fori_loop(0, ngrp, grp, jnp.zeros((_LANES,), jnp.int32))
            any_unk = plsc.all_reduce_population_count(acc != 0)[0] > 0

            cps = [
                pltpu.async_copy(
                    table_hbm.at[ids_v.at[j]],
                    rows_v.at[pl.ds(j * _IDXW, _IDXW)],
                    sem,
                )
                for j in range(ndma)
            ]
            for cp in cps:
                cp.wait()

            @pl.when(any_unk)
            def _fixup():
                def fix(j, c):
                    g = flag_v[pl.ds(j * _LANES, _LANES)]

                    @pl.when(plsc.all_reduce_population_count(g != 0)[0] > 0)
                    def _():
                        for lane in range(_LANES):
                            @pl.when(g[lane] == 1)
                            def _():
                                r = j * _LANES + lane
                                for q in range(d // _LANES):
                                    rows_v[r, pl.ds(q * _LANES, _LANES)] = (
                                        unk_v[pl.ds(q * _LANES, _LANES)])
                    return c
                lax.fori_loop(0, ngrp, fix, 0)

            pltpu.sync_copy(rows_v, out_hbm.at[pl.ds(off, chunk)])
            return carry

        lax.fori_loop(0, nchunk, chunk_body, 0)

    return emb


def kernel(grid_ids, special_pos, table, unk_emb):
    b, l = grid_ids.shape
    _, d = table.shape
    n = b * l
    emb = _build(n, d, 1024)
    gids = grid_ids.reshape(n)
    spec = special_pos.reshape(n).astype(jnp.int32)
    out = emb(gids, spec, table, unk_emb)
    return out.reshape(b, l, d)


# speedup vs baseline: 11.0683x; 5.5960x over previous
import functools

import jax
import jax.numpy as jnp
from jax import lax
from jax.experimental import pallas as pl
from jax.experimental.pallas import tpu as pltpu
from jax.experimental.pallas import tpu_sc as plsc

_UNKNOWN = 1
_LANES = 16
_IDXW = 128


@functools.lru_cache(maxsize=None)
def _build(n, d, chunk):
    info = plsc.get_sparse_core_info()
    nw = info.num_cores * info.num_subcores
    assert n % (nw * chunk) == 0
    per_w = n // nw
    nchunk = per_w // chunk
    ndma = chunk // _IDXW
    ngrp = chunk // _LANES
    gpr = _IDXW // _LANES

    mesh = plsc.VectorSubcoreMesh(core_axis_name="c", subcore_axis_name="s")

    @functools.partial(
        pl.kernel,
        mesh=mesh,
        compiler_params=pltpu.CompilerParams(
            needs_layout_passes=False, use_tc_tiling_on_sc=False),
        out_type=jax.ShapeDtypeStruct((n, d), jnp.float32),
        scratch_types=[
            pltpu.VMEM((ndma, _IDXW), jnp.int32),
            pltpu.VMEM((ndma, _IDXW), jnp.int32),
            pltpu.VMEM((ndma, _IDXW), jnp.int32),
            pltpu.VMEM((chunk, d), jnp.float32),
            pltpu.VMEM((d,), jnp.float32),
            pltpu.VMEM((d,), jnp.float32),
            pltpu.SemaphoreType.DMA,
        ],
    )
    def emb(gids_hbm, spec_hbm, table_hbm, unk_hbm, out_hbm,
            ids_v, spec_v, uflag_v, rows_v, t0_v, uemb_v, sem):
        wid = lax.axis_index("s") * info.num_cores + lax.axis_index("c")
        base = wid * per_w
        pltpu.sync_copy(unk_hbm, uemb_v)
        pltpu.sync_copy(table_hbm.at[0], t0_v)
        t0q = [t0_v[pl.ds(q * _LANES, _LANES)] for q in range(d // _LANES)]

        def chunk_body(ci, carry):
            off = base + ci * chunk
            row_off = off // _IDXW
            pltpu.sync_copy(gids_hbm.at[pl.ds(row_off, ndma)], ids_v)
            pltpu.sync_copy(spec_hbm.at[pl.ds(row_off, ndma)], spec_v)

            cps = [
                pltpu.async_copy(
                    table_hbm.at[ids_v.at[j]],
                    rows_v.at[pl.ds(j * _IDXW, _IDXW)],
                    sem,
                )
                for j in range(ndma)
            ]

            def grp(j, acc):
                sl = pl.ds((j % gpr) * _LANES, _LANES)
                g = ids_v[j // gpr, sl]
                unk = g == _UNKNOWN
                unk_i = jnp.where(unk, 1, 0)
                uflag_v[j // gpr, sl] = unk_i
                return acc | unk_i

            acc = lax.fori_loop(0, ngrp, grp, jnp.zeros((_LANES,), jnp.int32))
            any_unk = plsc.all_reduce_population_count(acc != 0)[0] > 0

            for cp in cps:
                cp.wait()

            def blend(j, c):
                sv = spec_v[j // gpr, pl.ds((j % gpr) * _LANES, _LANES)]
                for lane in range(_LANES):
                    r = j * _LANES + lane
                    m = jnp.broadcast_to(sv[lane], (_LANES,)) != 0
                    for q in range(d // _LANES):
                        sl = pl.ds(q * _LANES, _LANES)
                        rows_v[r, sl] = jnp.where(m, t0q[q], rows_v[r, sl])
                return c
            lax.fori_loop(0, ngrp, blend, 0)

            @pl.when(any_unk)
            def _fixup():
                def fix(j, c):
                    sl = pl.ds((j % gpr) * _LANES, _LANES)
                    g = uflag_v[j // gpr, sl]

                    @pl.when(plsc.all_reduce_population_count(g != 0)[0] > 0)
                    def _():
                        for lane in range(_LANES):
                            @pl.when(g[lane] == 1)
                            def _():
                                r = j * _LANES + lane
                                for q in range(d // _LANES):
                                    rows_v[r, pl.ds(q * _LANES, _LANES)] = (
                                        uemb_v[pl.ds(q * _LANES, _LANES)])
                    return c
                lax.fori_loop(0, ngrp, fix, 0)

            pltpu.sync_copy(rows_v, out_hbm.at[pl.ds(off, chunk)])
            return carry

        lax.fori_loop(0, nchunk, chunk_body, 0)

    return emb


def kernel(grid_ids, special_pos, table, unk_emb):
    b, l = grid_ids.shape
    _, d = table.shape
    n = b * l
    emb = _build(n, d, 1024)
    gids = grid_ids.reshape(n // _IDXW, _IDXW)
    spec = special_pos.reshape(n // _IDXW, _IDXW).astype(jnp.int32)
    out = emb(gids, spec, table, unk_emb)
    return out.reshape(b, l, d)


# speedup vs baseline: 13.1383x; 1.1870x over previous
import functools

import jax
import jax.numpy as jnp
from jax import lax
from jax.experimental import pallas as pl
from jax.experimental.pallas import tpu as pltpu
from jax.experimental.pallas import tpu_sc as plsc

_UNKNOWN = 1
_LANES = 16


def _windows(l):
    out = []
    pos = 0
    while pos + _LANES <= l:
        out.append((pos, 0))
        pos += _LANES
    if pos < l:
        out.append((l - _LANES, _LANES - (l - pos)))
    return out


@functools.lru_cache(maxsize=None)
def _build(b, l, d, nb):
    info = plsc.get_sparse_core_info()
    nw = info.num_cores * info.num_subcores
    assert b % (nw * nb) == 0
    rows_w = b // nw
    nchunk = rows_w // nb
    wins = _windows(l)
    nq = d // _LANES

    mesh = plsc.VectorSubcoreMesh(core_axis_name="c", subcore_axis_name="s")

    @functools.partial(
        pl.kernel,
        mesh=mesh,
        compiler_params=pltpu.CompilerParams(
            needs_layout_passes=False, use_tc_tiling_on_sc=False),
        out_type=jax.ShapeDtypeStruct((b, l, d), jnp.float32),
        scratch_types=[
            pltpu.VMEM((nb, l), jnp.int32),
            pltpu.VMEM((nb, l), jnp.int32),
            pltpu.VMEM((nb, l), jnp.int32),
            pltpu.VMEM((nb, l, d), jnp.float32),
            pltpu.VMEM((d,), jnp.float32),
            pltpu.VMEM((d,), jnp.float32),
            pltpu.SemaphoreType.DMA,
        ],
    )
    def emb(gids_hbm, spec_hbm, table_hbm, unk_hbm, out_hbm,
            gids_v, spec_v, uflag_v, rows_v, t0_v, uemb_v, sem):
        wid = lax.axis_index("s") * info.num_cores + lax.axis_index("c")
        base = wid * rows_w
        pltpu.sync_copy(unk_hbm, uemb_v)
        pltpu.sync_copy(table_hbm.at[0], t0_v)
        t0q = [t0_v[pl.ds(q * _LANES, _LANES)] for q in range(nq)]

        def chunk_body(ci, carry):
            boff = base + ci * nb
            pltpu.sync_copy(gids_hbm.at[pl.ds(boff, nb)], gids_v)
            pltpu.sync_copy(spec_hbm.at[pl.ds(boff, nb)], spec_v)

            cps = [
                pltpu.async_copy(
                    table_hbm.at[gids_v.at[i]], rows_v.at[i], sem)
                for i in range(nb)
            ]

            def detect(i, acc):
                for off, _ in wins:
                    g = gids_v[i, pl.ds(off, _LANES)]
                    unk_i = jnp.where(g == _UNKNOWN, 1, 0)
                    uflag_v[i, pl.ds(off, _LANES)] = unk_i
                    acc = acc | unk_i
                return acc

            acc = lax.fori_loop(0, nb, detect,
                                jnp.zeros((_LANES,), jnp.int32))
            any_unk = plsc.all_reduce_population_count(acc != 0)[0] > 0

            for cp in cps:
                cp.wait()

            def blend(i, c):
                for off, lane0 in wins:
                    sv = spec_v[i, pl.ds(off, _LANES)]
                    for lane in range(lane0, _LANES):
                        j = off + lane
                        m = jnp.broadcast_to(sv[lane], (_LANES,)) != 0
                        for q in range(nq):
                            sl = pl.ds(q * _LANES, _LANES)
                            rows_v[i, j, sl] = jnp.where(
                                m, t0q[q], rows_v[i, j, sl])
                return c
            lax.fori_loop(0, nb, blend, 0)

            @pl.when(any_unk)
            def _fixup():
                def fix(i, c):
                    for off, lane0 in wins:
                        g = uflag_v[i, pl.ds(off, _LANES)]

                        @pl.when(plsc.all_reduce_population_count(
                            g != 0)[0] > 0)
                        def _():
                            for lane in range(lane0, _LANES):
                                @pl.when(g[lane] == 1)
                                def _():
                                    j = off + lane
                                    for q in range(nq):
                                        sl = pl.ds(q * _LANES, _LANES)
                                        rows_v[i, j, sl] = uemb_v[sl]
                    return c
                lax.fori_loop(0, nb, fix, 0)

            pltpu.sync_copy(rows_v, out_hbm.at[pl.ds(boff, nb)])
            return carry

        lax.fori_loop(0, nchunk, chunk_body, 0)

    return emb


def kernel(grid_ids, special_pos, table, unk_emb):
    b, l = grid_ids.shape
    _, d = table.shape
    emb = _build(b, l, d, 16)
    spec = special_pos.astype(jnp.int32)
    return emb(grid_ids, spec, table, unk_emb)
